# Initial kernel scaffold; baseline (speedup 1.0000x reference)
import functools

import jax
import jax.numpy as jnp
from jax import lax
from jax.experimental import pallas as pl
from jax.experimental.pallas import tpu as pltpu
from jax.experimental.pallas import tpu_sc as plsc

N = 10000
E = 320000
F_IN = 128
F_OUT = 16
H = 8
D = H * F_OUT

NP = 10016
NTILES = 32
CHUNK = 128
CHUNKS_PER_TILE = 79
EDGES_PER_TILE = CHUNK * CHUNKS_PER_TILE
EP = EDGES_PER_TILE * NTILES
ROWS_PER_TILE = NP // 16

_f32 = jnp.float32


def _proj_body(h_ref, wt_ref, b_ref, af_src_ref, af_tgt_ref, g2_ref,
               hp_ref, ss_ref, st_ref):
    hp = jnp.dot(h_ref[...], wt_ref[...],
                 preferred_element_type=_f32) + b_ref[...]
    hp_ref[0:N, :] = hp
    hp_ref[N:NP, :] = jnp.zeros((NP - N, D), _f32)
    g2 = g2_ref[...]
    ss_ref[0:N, :] = jnp.dot(hp * af_src_ref[...], g2,
                             preferred_element_type=_f32)
    st_ref[0:N, :] = jnp.dot(hp * af_tgt_ref[...], g2,
                             preferred_element_type=_f32)
    pad = jnp.full((NP - N, 16), -1e30, _f32)
    ss_ref[N:NP, :] = pad
    st_ref[N:NP, :] = pad


def _rdenom_body(dp_ref, rd_ref):
    rd_ref[...] = 1.0 / (dp_ref[0] + dp_ref[1] + 1e-16)


def _combine_body(op_ref, out_ref):
    out_ref[...] = op_ref[0, 0:N, :] + op_ref[1, 0:N, :]


def _pass1_body(src_hbm, tgt_hbm, ss_hbm, st_hbm, z16_hbm,
                expe_hbm, dp_hbm,
                sidx, tidx, srows, trows, denom_sh, sem):
    c = lax.axis_index("c")
    s = lax.axis_index("s")
    wid = c * 16 + s
    pltpu.sync_copy(z16_hbm.at[pl.ds(s * ROWS_PER_TILE, ROWS_PER_TILE)],
                    denom_sh.at[pl.ds(s * ROWS_PER_TILE, ROWS_PER_TILE)])
    plsc.subcore_barrier()
    base_t = wid * EDGES_PER_TILE

    @pl.loop(0, CHUNKS_PER_TILE)
    def _chunks(j):
        base = base_t + j * CHUNK
        pltpu.sync_copy(src_hbm.at[pl.ds(base, CHUNK)], sidx.at[0])
        pltpu.sync_copy(tgt_hbm.at[pl.ds(base, CHUNK)], tidx.at[0])
        pltpu.async_copy(ss_hbm.at[sidx.at[0]], srows, sem).wait()
        pltpu.async_copy(st_hbm.at[tidx.at[0]], trows, sem).wait()

        @pl.loop(0, CHUNK)
        def _edges(i):
            w = srows[i] + trows[i]
            srows[i] = jnp.exp(jnp.maximum(w, w * 0.2))

        pltpu.sync_copy(srows, denom_sh.at[tidx.at[0]], add=True)
        pltpu.sync_copy(srows, expe_hbm.at[pl.ds(base, CHUNK)])

    plsc.subcore_barrier()

    @pl.when(s == 0)
    def _dump():
        pltpu.sync_copy(denom_sh, dp_hbm.at[c])


def _pass2_body(src_hbm, tgt_hbm, hp_hbm, rd_hbm, expe_hbm, z128_hbm,
                op_hbm,
                sidx, tidx, hrows, rrows, erows, out_sh, sem):
    c = lax.axis_index("c")
    s = lax.axis_index("s")
    wid = c * 16 + s
    pltpu.sync_copy(z128_hbm.at[pl.ds(s * ROWS_PER_TILE, ROWS_PER_TILE)],
                    out_sh.at[pl.ds(s * ROWS_PER_TILE, ROWS_PER_TILE)])
    plsc.subcore_barrier()
    base_t = wid * EDGES_PER_TILE

    @pl.loop(0, CHUNKS_PER_TILE)
    def _chunks(j):
        base = base_t + j * CHUNK
        pltpu.sync_copy(src_hbm.at[pl.ds(base, CHUNK)], sidx.at[0])
        pltpu.sync_copy(tgt_hbm.at[pl.ds(base, CHUNK)], tidx.at[0])
        pltpu.async_copy(hp_hbm.at[sidx.at[0]], hrows, sem).wait()
        pltpu.async_copy(rd_hbm.at[tidx.at[0]], rrows, sem).wait()
        pltpu.sync_copy(expe_hbm.at[pl.ds(base, CHUNK)], erows)

        @pl.loop(0, CHUNK)
        def _edges(i):
            erows[i] = erows[i] * rrows[i]
            for h in range(H):
                bc = plsc.load_gather(
                    erows,
                    [jnp.full((16,), i, jnp.int32),
                     jnp.full((16,), h, jnp.int32)])
                sl = pl.ds(h * F_OUT, F_OUT)
                hrows[i, sl] = hrows[i, sl] * bc

        pltpu.sync_copy(hrows, out_sh.at[tidx.at[0]], add=True)

    plsc.subcore_barrier()
    row0 = s * ROWS_PER_TILE
    pltpu.sync_copy(out_sh.at[pl.ds(row0, ROWS_PER_TILE)],
                    op_hbm.at[c].at[pl.ds(row0, ROWS_PER_TILE)])


_MESH = plsc.VectorSubcoreMesh(core_axis_name="c", subcore_axis_name="s",
                               num_cores=2, num_subcores=16)

_pass1 = functools.partial(
    pl.kernel,
    out_type=(jax.ShapeDtypeStruct((EP, 16), _f32),
              jax.ShapeDtypeStruct((2, NP, 16), _f32)),
    mesh=_MESH,
    scratch_types=[
        pltpu.VMEM((1, CHUNK), jnp.int32),
        pltpu.VMEM((1, CHUNK), jnp.int32),
        pltpu.VMEM((CHUNK, 16), _f32),
        pltpu.VMEM((CHUNK, 16), _f32),
        pltpu.VMEM_SHARED((NP, 16), _f32),
        pltpu.SemaphoreType.DMA,
    ])(_pass1_body)

_pass2 = functools.partial(
    pl.kernel,
    out_type=jax.ShapeDtypeStruct((2, NP, D), _f32),
    mesh=_MESH,
    scratch_types=[
        pltpu.VMEM((1, CHUNK), jnp.int32),
        pltpu.VMEM((1, CHUNK), jnp.int32),
        pltpu.VMEM((CHUNK, D), _f32),
        pltpu.VMEM((CHUNK, 16), _f32),
        pltpu.VMEM((CHUNK, 16), _f32),
        pltpu.VMEM_SHARED((NP, D), _f32),
        pltpu.SemaphoreType.DMA,
    ])(_pass2_body)


def kernel(h_in, edge_index, W, b, a_src, a_tgt):
    src = edge_index[0].astype(jnp.int32)
    tgt = edge_index[1].astype(jnp.int32)
    pad = jnp.full((EP - E,), N, jnp.int32)
    src_p = jnp.concatenate([src, pad])
    tgt_p = jnp.concatenate([tgt, pad])

    wt = W.T
    b2 = b.reshape(1, D)
    af_src = a_src.reshape(1, D)
    af_tgt = a_tgt.reshape(1, D)
    j = jnp.arange(D) // F_OUT
    g = (j[:, None] == jnp.arange(H)[None, :]).astype(_f32)
    g2 = jnp.concatenate([g, g], axis=1)

    hp, ss, st = pl.pallas_call(
        _proj_body,
        out_shape=(jax.ShapeDtypeStruct((NP, D), _f32),
                   jax.ShapeDtypeStruct((NP, 16), _f32),
                   jax.ShapeDtypeStruct((NP, 16), _f32)),
    )(h_in, wt, b2, af_src, af_tgt, g2)

    z16 = jnp.zeros((NP, 16), _f32)
    z128 = jnp.zeros((NP, D), _f32)

    expe, dp = _pass1(src_p, tgt_p, ss, st, z16)

    rd = pl.pallas_call(
        _rdenom_body,
        out_shape=jax.ShapeDtypeStruct((NP, 16), _f32),
    )(dp)

    op = _pass2(src_p, tgt_p, hp, rd, expe, z128)

    out = pl.pallas_call(
        _combine_body,
        out_shape=jax.ShapeDtypeStruct((N, D), _f32),
    )(op)
    return out


# speedup vs baseline: 27.6377x; 27.6377x over previous
import functools

import jax
import jax.numpy as jnp
from jax import lax
from jax.experimental import pallas as pl
from jax.experimental.pallas import tpu as pltpu
from jax.experimental.pallas import tpu_sc as plsc

N = 10000
E = 320000
F_IN = 128
F_OUT = 16
H = 8
D = H * F_OUT

NP = 10112
NTILES = 32
CHUNK = 128
CHUNKS_PER_TILE = 79
EDGES_PER_TILE = CHUNK * CHUNKS_PER_TILE
EP = EDGES_PER_TILE * NTILES
ROWS_PER_TILE = NP // 16

_f32 = jnp.float32


def _proj_body(h_ref, wt_ref, b_ref, af_src_ref, af_tgt_ref, g2_ref,
               hp_ref, ss_ref, st_ref):
    hp = jnp.dot(h_ref[...], wt_ref[...],
                 preferred_element_type=_f32) + b_ref[...]
    hp_ref[0:N, :] = hp
    hp_ref[N:NP, :] = jnp.zeros((NP - N, D), _f32)
    g2 = g2_ref[...]
    ss_ref[0:N, :] = jnp.dot(hp * af_src_ref[...], g2,
                             preferred_element_type=_f32)
    st_ref[0:N, :] = jnp.dot(hp * af_tgt_ref[...], g2,
                             preferred_element_type=_f32)
    pad = jnp.full((NP - N, 16), -1e30, _f32)
    ss_ref[N:NP, :] = pad
    st_ref[N:NP, :] = pad


def _rdenom_body(dp_ref, rd_ref):
    rd_ref[...] = 1.0 / (dp_ref[0] + dp_ref[1] + 1e-16)


def _combine_body(op_ref, out_ref):
    out_ref[...] = op_ref[0, 0:N, :] + op_ref[1, 0:N, :]


def _pass1_body(src_hbm, tgt_hbm, ss_hbm, st_hbm, z16_hbm,
                expe_hbm, dp_hbm,
                sidx, tidx, srows, trows, denom_sh, sem):
    c = lax.axis_index("c")
    s = lax.axis_index("s")
    wid = c * 16 + s
    pltpu.sync_copy(z16_hbm.at[pl.ds(s * ROWS_PER_TILE, ROWS_PER_TILE)],
                    denom_sh.at[pl.ds(s * ROWS_PER_TILE, ROWS_PER_TILE)])
    plsc.subcore_barrier()
    base_t = wid * EDGES_PER_TILE

    @pl.loop(0, CHUNKS_PER_TILE)
    def _chunks(j):
        base = base_t + j * CHUNK
        pltpu.sync_copy(src_hbm.at[pl.ds(base, CHUNK)], sidx.at[0])
        pltpu.sync_copy(tgt_hbm.at[pl.ds(base, CHUNK)], tidx.at[0])
        pltpu.async_copy(ss_hbm.at[sidx.at[0]], srows, sem).wait()
        pltpu.async_copy(st_hbm.at[tidx.at[0]], trows, sem).wait()

        @pl.loop(0, CHUNK)
        def _edges(i):
            w = srows[i] + trows[i]
            srows[i] = jnp.exp(jnp.maximum(w, w * 0.2))

        pltpu.sync_copy(srows, denom_sh.at[tidx.at[0]], add=True)
        pltpu.sync_copy(srows, expe_hbm.at[pl.ds(base, CHUNK)])

    plsc.subcore_barrier()

    @pl.when(s == 0)
    def _dump():
        pltpu.sync_copy(denom_sh, dp_hbm.at[c])


def _pass2_body(src_hbm, tgt_hbm, hp_hbm, rd_hbm, expe_hbm, z128_hbm,
                op_hbm,
                sidx, tidx, hrows, rrows, erows, out_sh, sem):
    c = lax.axis_index("c")
    s = lax.axis_index("s")
    wid = c * 16 + s
    pltpu.sync_copy(z128_hbm.at[pl.ds(s * ROWS_PER_TILE, ROWS_PER_TILE)],
                    out_sh.at[pl.ds(s * ROWS_PER_TILE, ROWS_PER_TILE)])
    plsc.subcore_barrier()
    base_t = wid * EDGES_PER_TILE

    @pl.loop(0, CHUNKS_PER_TILE)
    def _chunks(j):
        base = base_t + j * CHUNK
        pltpu.sync_copy(src_hbm.at[pl.ds(base, CHUNK)], sidx.at[0])
        pltpu.sync_copy(tgt_hbm.at[pl.ds(base, CHUNK)], tidx.at[0])
        pltpu.async_copy(hp_hbm.at[sidx.at[0]], hrows, sem).wait()
        pltpu.async_copy(rd_hbm.at[tidx.at[0]], rrows, sem).wait()
        pltpu.sync_copy(expe_hbm.at[pl.ds(base, CHUNK)], erows)

        @pl.loop(0, CHUNK)
        def _edges(i):
            erows[i] = erows[i] * rrows[i]
            for h in range(H):
                bc = plsc.load_gather(
                    erows,
                    [jnp.full((16,), i, jnp.int32),
                     jnp.full((16,), h, jnp.int32)])
                sl = pl.ds(h * F_OUT, F_OUT)
                hrows[i, sl] = hrows[i, sl] * bc

        pltpu.sync_copy(hrows, out_sh.at[tidx.at[0]], add=True)

    plsc.subcore_barrier()
    row0 = s * ROWS_PER_TILE
    pltpu.sync_copy(out_sh.at[pl.ds(row0, ROWS_PER_TILE)],
                    op_hbm.at[c].at[pl.ds(row0, ROWS_PER_TILE)])


_MESH = plsc.VectorSubcoreMesh(core_axis_name="c", subcore_axis_name="s",
                               num_cores=2, num_subcores=16)

_SC_PARAMS = pltpu.CompilerParams(use_tc_tiling_on_sc=False,
                                  needs_layout_passes=False)

_pass1 = functools.partial(
    pl.kernel,
    out_type=(jax.ShapeDtypeStruct((EP, 16), _f32),
              jax.ShapeDtypeStruct((2, NP, 16), _f32)),
    mesh=_MESH,
    scratch_types=[
        pltpu.VMEM((1, CHUNK), jnp.int32),
        pltpu.VMEM((1, CHUNK), jnp.int32),
        pltpu.VMEM((CHUNK, 16), _f32),
        pltpu.VMEM((CHUNK, 16), _f32),
        pltpu.VMEM_SHARED((NP, 16), _f32),
        pltpu.SemaphoreType.DMA,
    ],
    compiler_params=_SC_PARAMS)(_pass1_body)

_pass2 = functools.partial(
    pl.kernel,
    out_type=jax.ShapeDtypeStruct((2, NP, D), _f32),
    mesh=_MESH,
    scratch_types=[
        pltpu.VMEM((1, CHUNK), jnp.int32),
        pltpu.VMEM((1, CHUNK), jnp.int32),
        pltpu.VMEM((CHUNK, D), _f32),
        pltpu.VMEM((CHUNK, 16), _f32),
        pltpu.VMEM((CHUNK, 16), _f32),
        pltpu.VMEM_SHARED((NP, D), _f32),
        pltpu.SemaphoreType.DMA,
    ],
    compiler_params=_SC_PARAMS)(_pass2_body)


def kernel(h_in, edge_index, W, b, a_src, a_tgt):
    src = edge_index[0].astype(jnp.int32)
    tgt = edge_index[1].astype(jnp.int32)
    pad = jnp.full((EP - E,), N, jnp.int32)
    src_p = jnp.concatenate([src, pad])
    tgt_p = jnp.concatenate([tgt, pad])

    wt = W.T
    b2 = b.reshape(1, D)
    af_src = a_src.reshape(1, D)
    af_tgt = a_tgt.reshape(1, D)
    j = jnp.arange(D) // F_OUT
    g = (j[:, None] == jnp.arange(H)[None, :]).astype(_f32)
    g2 = jnp.concatenate([g, g], axis=1)

    hp, ss, st = pl.pallas_call(
        _proj_body,
        out_shape=(jax.ShapeDtypeStruct((NP, D), _f32),
                   jax.ShapeDtypeStruct((NP, 16), _f32),
                   jax.ShapeDtypeStruct((NP, 16), _f32)),
    )(h_in, wt, b2, af_src, af_tgt, g2)

    z16 = jnp.zeros((NP, 16), _f32)
    z128 = jnp.zeros((NP, D), _f32)

    expe, dp = _pass1(src_p, tgt_p, ss, st, z16)

    rd = pl.pallas_call(
        _rdenom_body,
        out_shape=jax.ShapeDtypeStruct((NP, 16), _f32),
    )(dp)

    op = _pass2(src_p, tgt_p, hp, rd, expe, z128)

    out = pl.pallas_call(
        _combine_body,
        out_shape=jax.ShapeDtypeStruct((N, D), _f32),
    )(op)
    return out


# speedup vs baseline: 43.9330x; 1.5896x over previous
import functools

import jax
import jax.numpy as jnp
from jax import lax
from jax.experimental import pallas as pl
from jax.experimental.pallas import tpu as pltpu
from jax.experimental.pallas import tpu_sc as plsc

N = 10000
E = 320000
F_IN = 128
F_OUT = 16
H = 8
D = H * F_OUT

NP = 10112
NTILES = 32
CHUNK = 128
CHUNKS_PER_TILE = 79
EDGES_PER_TILE = CHUNK * CHUNKS_PER_TILE
EP = EDGES_PER_TILE * NTILES
ROWS_PER_TILE = NP // 16

_f32 = jnp.float32

_GDN = lax.GatherDimensionNumbers(offset_dims=(), collapsed_slice_dims=(0,),
                                  start_index_map=(0,))


def _bcast(vec, h):
    idx = jnp.full((16, 1), h, jnp.int32)
    return lax.gather(vec, idx, _GDN, slice_sizes=(1,),
                      mode=lax.GatherScatterMode.PROMISE_IN_BOUNDS)


def _proj_body(h_ref, wt_ref, b_ref, af_src_ref, af_tgt_ref, g2_ref,
               hp_ref, ss_ref, st_ref):
    hp = jnp.dot(h_ref[...], wt_ref[...],
                 preferred_element_type=_f32) + b_ref[...]
    hp_ref[0:N, :] = hp
    hp_ref[N:NP, :] = jnp.zeros((NP - N, D), _f32)
    g2 = g2_ref[...]
    ss_ref[0:N, :] = jnp.dot(hp * af_src_ref[...], g2,
                             preferred_element_type=_f32)
    st_ref[0:N, :] = jnp.dot(hp * af_tgt_ref[...], g2,
                             preferred_element_type=_f32)
    pad = jnp.full((NP - N, 16), -1e30, _f32)
    ss_ref[N:NP, :] = pad
    st_ref[N:NP, :] = pad


def _rdenom_body(dp_ref, rd_ref):
    rd_ref[...] = 1.0 / (dp_ref[0] + dp_ref[1] + 1e-16)


def _combine_body(op_ref, out_ref):
    out_ref[...] = op_ref[0, 0:N, :] + op_ref[1, 0:N, :]


def _pass1_body(src_hbm, tgt_hbm, ss_hbm, st_hbm, z16_hbm,
                expe_hbm, dp_hbm,
                sidx, tidx, srows, trows, denom_sh, sem):
    c = lax.axis_index("c")
    s = lax.axis_index("s")
    wid = c * 16 + s
    pltpu.sync_copy(z16_hbm.at[pl.ds(s * ROWS_PER_TILE, ROWS_PER_TILE)],
                    denom_sh.at[pl.ds(s * ROWS_PER_TILE, ROWS_PER_TILE)])
    plsc.subcore_barrier()
    base_t = wid * EDGES_PER_TILE

    @pl.loop(0, CHUNKS_PER_TILE)
    def _chunks(j):
        base = base_t + j * CHUNK
        pltpu.sync_copy(src_hbm.at[pl.ds(base, CHUNK)], sidx.at[0])
        pltpu.sync_copy(tgt_hbm.at[pl.ds(base, CHUNK)], tidx.at[0])
        pltpu.async_copy(ss_hbm.at[sidx.at[0]], srows, sem).wait()
        pltpu.async_copy(st_hbm.at[tidx.at[0]], trows, sem).wait()

        @plsc.parallel_loop(0, CHUNK, unroll=8)
        def _edges(i):
            w = srows[i] + trows[i]
            srows[i] = jnp.exp(jnp.maximum(w, w * 0.2))

        pltpu.sync_copy(srows, denom_sh.at[tidx.at[0]], add=True)
        pltpu.sync_copy(srows, expe_hbm.at[pl.ds(base, CHUNK)])

    plsc.subcore_barrier()

    @pl.when(s == 0)
    def _dump():
        pltpu.sync_copy(denom_sh, dp_hbm.at[c])


def _pass2_body(src_hbm, tgt_hbm, hp_hbm, rd_hbm, expe_hbm, z128_hbm,
                op_hbm,
                sidx, tidx, hrows, rrows, erows, out_sh, sem):
    c = lax.axis_index("c")
    s = lax.axis_index("s")
    wid = c * 16 + s
    pltpu.sync_copy(z128_hbm.at[pl.ds(s * ROWS_PER_TILE, ROWS_PER_TILE)],
                    out_sh.at[pl.ds(s * ROWS_PER_TILE, ROWS_PER_TILE)])
    plsc.subcore_barrier()
    base_t = wid * EDGES_PER_TILE

    @pl.loop(0, CHUNKS_PER_TILE)
    def _chunks(j):
        base = base_t + j * CHUNK
        pltpu.sync_copy(src_hbm.at[pl.ds(base, CHUNK)], sidx.at[0])
        pltpu.sync_copy(tgt_hbm.at[pl.ds(base, CHUNK)], tidx.at[0])
        pltpu.async_copy(hp_hbm.at[sidx.at[0]], hrows, sem).wait()
        pltpu.async_copy(rd_hbm.at[tidx.at[0]], rrows, sem).wait()
        pltpu.sync_copy(expe_hbm.at[pl.ds(base, CHUNK)], erows)

        @plsc.parallel_loop(0, CHUNK, unroll=4)
        def _edges(i):
            alpha = erows[i] * rrows[i]
            for h in range(H):
                sl = pl.ds(h * F_OUT, F_OUT)
                hrows[i, sl] = hrows[i, sl] * _bcast(alpha, h)

        pltpu.sync_copy(hrows, out_sh.at[tidx.at[0]], add=True)

    plsc.subcore_barrier()
    row0 = s * ROWS_PER_TILE
    pltpu.sync_copy(out_sh.at[pl.ds(row0, ROWS_PER_TILE)],
                    op_hbm.at[c].at[pl.ds(row0, ROWS_PER_TILE)])


_MESH = plsc.VectorSubcoreMesh(core_axis_name="c", subcore_axis_name="s",
                               num_cores=2, num_subcores=16)

_SC_PARAMS = pltpu.CompilerParams(use_tc_tiling_on_sc=False,
                                  needs_layout_passes=False)

_pass1 = functools.partial(
    pl.kernel,
    out_type=(jax.ShapeDtypeStruct((EP, 16), _f32),
              jax.ShapeDtypeStruct((2, NP, 16), _f32)),
    mesh=_MESH,
    scratch_types=[
        pltpu.VMEM((1, CHUNK), jnp.int32),
        pltpu.VMEM((1, CHUNK), jnp.int32),
        pltpu.VMEM((CHUNK, 16), _f32),
        pltpu.VMEM((CHUNK, 16), _f32),
        pltpu.VMEM_SHARED((NP, 16), _f32),
        pltpu.SemaphoreType.DMA,
    ],
    compiler_params=_SC_PARAMS)(_pass1_body)

_pass2 = functools.partial(
    pl.kernel,
    out_type=jax.ShapeDtypeStruct((2, NP, D), _f32),
    mesh=_MESH,
    scratch_types=[
        pltpu.VMEM((1, CHUNK), jnp.int32),
        pltpu.VMEM((1, CHUNK), jnp.int32),
        pltpu.VMEM((CHUNK, D), _f32),
        pltpu.VMEM((CHUNK, 16), _f32),
        pltpu.VMEM((CHUNK, 16), _f32),
        pltpu.VMEM_SHARED((NP, D), _f32),
        pltpu.SemaphoreType.DMA,
    ],
    compiler_params=_SC_PARAMS)(_pass2_body)


def kernel(h_in, edge_index, W, b, a_src, a_tgt):
    src = edge_index[0].astype(jnp.int32)
    tgt = edge_index[1].astype(jnp.int32)
    pad = jnp.full((EP - E,), N, jnp.int32)
    src_p = jnp.concatenate([src, pad])
    tgt_p = jnp.concatenate([tgt, pad])

    wt = W.T
    b2 = b.reshape(1, D)
    af_src = a_src.reshape(1, D)
    af_tgt = a_tgt.reshape(1, D)
    j = jnp.arange(D) // F_OUT
    g = (j[:, None] == jnp.arange(H)[None, :]).astype(_f32)
    g2 = jnp.concatenate([g, g], axis=1)

    hp, ss, st = pl.pallas_call(
        _proj_body,
        out_shape=(jax.ShapeDtypeStruct((NP, D), _f32),
                   jax.ShapeDtypeStruct((NP, 16), _f32),
                   jax.ShapeDtypeStruct((NP, 16), _f32)),
    )(h_in, wt, b2, af_src, af_tgt, g2)

    z16 = jnp.zeros((NP, 16), _f32)
    z128 = jnp.zeros((NP, D), _f32)

    expe, dp = _pass1(src_p, tgt_p, ss, st, z16)

    rd = pl.pallas_call(
        _rdenom_body,
        out_shape=jax.ShapeDtypeStruct((NP, 16), _f32),
    )(dp)

    op = _pass2(src_p, tgt_p, hp, rd, expe, z128)

    out = pl.pallas_call(
        _combine_body,
        out_shape=jax.ShapeDtypeStruct((N, D), _f32),
    )(op)
    return out


# speedup vs baseline: 59.5333x; 1.3551x over previous
import functools

import jax
import jax.numpy as jnp
from jax import lax
from jax.experimental import pallas as pl
from jax.experimental.pallas import tpu as pltpu
from jax.experimental.pallas import tpu_sc as plsc

N = 10000
E = 320000
F_IN = 128
F_OUT = 16
H = 8
D = H * F_OUT

NP = 10112
NTILES = 32
CHUNK = 128
CHUNKS_PER_TILE = 80
C2 = 64
CPT2 = 160
EDGES_PER_TILE = CHUNK * CHUNKS_PER_TILE
EP = EDGES_PER_TILE * NTILES
ROWS_PER_TILE = NP // 16

_f32 = jnp.float32

_GDN = lax.GatherDimensionNumbers(offset_dims=(), collapsed_slice_dims=(0,),
                                  start_index_map=(0,))


def _bcast(vec, h):
    idx = jnp.full((16, 1), h, jnp.int32)
    return lax.gather(vec, idx, _GDN, slice_sizes=(1,),
                      mode=lax.GatherScatterMode.PROMISE_IN_BOUNDS)


def _proj_body(h_ref, wt_ref, b_ref, af_src_ref, af_tgt_ref, g2_ref,
               hp_ref, ss_ref, st_ref):
    hp = jnp.dot(h_ref[...], wt_ref[...],
                 preferred_element_type=_f32) + b_ref[...]
    hp_ref[0:N, :] = hp
    hp_ref[N:NP, :] = jnp.zeros((NP - N, D), _f32)
    g2 = g2_ref[...]
    ss_ref[0:N, :] = jnp.dot(hp * af_src_ref[...], g2,
                             preferred_element_type=_f32)
    st_ref[0:N, :] = jnp.dot(hp * af_tgt_ref[...], g2,
                             preferred_element_type=_f32)
    pad = jnp.full((NP - N, 16), -1e30, _f32)
    ss_ref[N:NP, :] = pad
    st_ref[N:NP, :] = pad


def _rdenom_body(dp_ref, rd_ref):
    rd_ref[...] = 1.0 / (dp_ref[0] + dp_ref[1] + 1e-16)


def _combine_body(op_ref, out_ref):
    out_ref[...] = op_ref[0, 0:N, :] + op_ref[1, 0:N, :]


def _pass1_body(src_hbm, tgt_hbm, ss_hbm, st_hbm, z16_hbm,
                expe_hbm, dp_hbm,
                sidx_all, tidx_all, srows0, trows0, srows1, trows1,
                denom_sh, sem0, sem1):
    c = lax.axis_index("c")
    s = lax.axis_index("s")
    wid = c * 16 + s
    pltpu.sync_copy(z16_hbm.at[pl.ds(s * ROWS_PER_TILE, ROWS_PER_TILE)],
                    denom_sh.at[pl.ds(s * ROWS_PER_TILE, ROWS_PER_TILE)])
    pltpu.sync_copy(src_hbm.at[pl.ds(wid * CHUNKS_PER_TILE, CHUNKS_PER_TILE)],
                    sidx_all)
    pltpu.sync_copy(tgt_hbm.at[pl.ds(wid * CHUNKS_PER_TILE, CHUNKS_PER_TILE)],
                    tidx_all)
    plsc.subcore_barrier()
    base_t = wid * EDGES_PER_TILE

    def _start(j, sr, tr, sem):
        pltpu.async_copy(ss_hbm.at[sidx_all.at[j]], sr, sem)
        pltpu.async_copy(st_hbm.at[tidx_all.at[j]], tr, sem)

    def _wait(j, sr, tr, sem):
        pltpu.make_async_copy(ss_hbm.at[sidx_all.at[j]], sr, sem).wait()
        pltpu.make_async_copy(st_hbm.at[tidx_all.at[j]], tr, sem).wait()

    def _work(j, sr, tr, sem):
        _wait(j, sr, tr, sem)

        @plsc.parallel_loop(0, CHUNK, unroll=8)
        def _edges(i):
            w = sr[i] + tr[i]
            sr[i] = jnp.exp(jnp.maximum(w, w * 0.2))

        pltpu.sync_copy(sr, denom_sh.at[tidx_all.at[j]], add=True)
        pltpu.sync_copy(sr, expe_hbm.at[pl.ds(base_t + j * CHUNK, CHUNK)])

        @pl.when(j + 2 < CHUNKS_PER_TILE)
        def _next():
            _start(j + 2, sr, tr, sem)

    _start(0, srows0, trows0, sem0)
    _start(1, srows1, trows1, sem1)

    @pl.loop(0, CHUNKS_PER_TILE, step=2)
    def _chunks(j):
        _work(j, srows0, trows0, sem0)
        _work(j + 1, srows1, trows1, sem1)

    plsc.subcore_barrier()

    @pl.when(s == 0)
    def _dump():
        pltpu.sync_copy(denom_sh, dp_hbm.at[c])


def _pass2_body(src_hbm, tgt_hbm, hp_hbm, rd_hbm, expe_hbm, z128_hbm,
                op_hbm,
                sidx_all, tidx_all, hrows0, rrows0, erows0,
                hrows1, rrows1, erows1, out_sh, sem0, sem1):
    c = lax.axis_index("c")
    s = lax.axis_index("s")
    wid = c * 16 + s
    pltpu.sync_copy(z128_hbm.at[pl.ds(s * ROWS_PER_TILE, ROWS_PER_TILE)],
                    out_sh.at[pl.ds(s * ROWS_PER_TILE, ROWS_PER_TILE)])
    pltpu.sync_copy(src_hbm.at[pl.ds(wid * CPT2, CPT2)], sidx_all)
    pltpu.sync_copy(tgt_hbm.at[pl.ds(wid * CPT2, CPT2)], tidx_all)
    plsc.subcore_barrier()
    base_t = wid * EDGES_PER_TILE

    def _start(j, hr, rr, er, sem):
        pltpu.async_copy(hp_hbm.at[sidx_all.at[j]], hr, sem)
        pltpu.async_copy(rd_hbm.at[tidx_all.at[j]], rr, sem)
        pltpu.async_copy(expe_hbm.at[pl.ds(base_t + j * C2, C2)],
                         er, sem)

    def _wait(j, hr, rr, er, sem):
        pltpu.make_async_copy(hp_hbm.at[sidx_all.at[j]], hr, sem).wait()
        pltpu.make_async_copy(rd_hbm.at[tidx_all.at[j]], rr, sem).wait()
        pltpu.make_async_copy(expe_hbm.at[pl.ds(base_t + j * C2, C2)],
                              er, sem).wait()

    def _work(j, hr, rr, er, sem):
        _wait(j, hr, rr, er, sem)

        @plsc.parallel_loop(0, C2, unroll=4)
        def _edges(i):
            alpha = er[i] * rr[i]
            for h in range(H):
                sl = pl.ds(h * F_OUT, F_OUT)
                hr[i, sl] = hr[i, sl] * _bcast(alpha, h)

        pltpu.sync_copy(hr, out_sh.at[tidx_all.at[j]], add=True)

        @pl.when(j + 2 < CPT2)
        def _next():
            _start(j + 2, hr, rr, er, sem)

    _start(0, hrows0, rrows0, erows0, sem0)
    _start(1, hrows1, rrows1, erows1, sem1)

    @pl.loop(0, CPT2, step=2)
    def _chunks(j):
        _work(j, hrows0, rrows0, erows0, sem0)
        _work(j + 1, hrows1, rrows1, erows1, sem1)

    plsc.subcore_barrier()
    row0 = s * ROWS_PER_TILE
    pltpu.sync_copy(out_sh.at[pl.ds(row0, ROWS_PER_TILE)],
                    op_hbm.at[c].at[pl.ds(row0, ROWS_PER_TILE)])


_MESH = plsc.VectorSubcoreMesh(core_axis_name="c", subcore_axis_name="s",
                               num_cores=2, num_subcores=16)

_SC_PARAMS = pltpu.CompilerParams(use_tc_tiling_on_sc=False,
                                  needs_layout_passes=False)

_pass1 = functools.partial(
    pl.kernel,
    out_type=(jax.ShapeDtypeStruct((EP, 16), _f32),
              jax.ShapeDtypeStruct((2, NP, 16), _f32)),
    mesh=_MESH,
    scratch_types=[
        pltpu.VMEM((CHUNKS_PER_TILE, CHUNK), jnp.int32),
        pltpu.VMEM((CHUNKS_PER_TILE, CHUNK), jnp.int32),
        pltpu.VMEM((CHUNK, 16), _f32),
        pltpu.VMEM((CHUNK, 16), _f32),
        pltpu.VMEM((CHUNK, 16), _f32),
        pltpu.VMEM((CHUNK, 16), _f32),
        pltpu.VMEM_SHARED((NP, 16), _f32),
        pltpu.SemaphoreType.DMA,
        pltpu.SemaphoreType.DMA,
    ],
    compiler_params=_SC_PARAMS)(_pass1_body)

_pass2 = functools.partial(
    pl.kernel,
    out_type=jax.ShapeDtypeStruct((2, NP, D), _f32),
    mesh=_MESH,
    scratch_types=[
        pltpu.VMEM((CPT2, C2), jnp.int32),
        pltpu.VMEM((CPT2, C2), jnp.int32),
        pltpu.VMEM((C2, D), _f32),
        pltpu.VMEM((C2, 16), _f32),
        pltpu.VMEM((C2, 16), _f32),
        pltpu.VMEM((C2, D), _f32),
        pltpu.VMEM((C2, 16), _f32),
        pltpu.VMEM((C2, 16), _f32),
        pltpu.VMEM_SHARED((NP, D), _f32),
        pltpu.SemaphoreType.DMA,
        pltpu.SemaphoreType.DMA,
    ],
    compiler_params=_SC_PARAMS)(_pass2_body)


def kernel(h_in, edge_index, W, b, a_src, a_tgt):
    src = edge_index[0].astype(jnp.int32)
    tgt = edge_index[1].astype(jnp.int32)
    pad = jnp.full((EP - E,), N, jnp.int32)
    src_flat = jnp.concatenate([src, pad])
    tgt_flat = jnp.concatenate([tgt, pad])
    src_p = src_flat.reshape(EP // CHUNK, CHUNK)
    tgt_p = tgt_flat.reshape(EP // CHUNK, CHUNK)
    src_p2 = src_flat.reshape(EP // C2, C2)
    tgt_p2 = tgt_flat.reshape(EP // C2, C2)

    wt = W.T
    b2 = b.reshape(1, D)
    af_src = a_src.reshape(1, D)
    af_tgt = a_tgt.reshape(1, D)
    j = jnp.arange(D) // F_OUT
    g = (j[:, None] == jnp.arange(H)[None, :]).astype(_f32)
    g2 = jnp.concatenate([g, g], axis=1)

    hp, ss, st = pl.pallas_call(
        _proj_body,
        out_shape=(jax.ShapeDtypeStruct((NP, D), _f32),
                   jax.ShapeDtypeStruct((NP, 16), _f32),
                   jax.ShapeDtypeStruct((NP, 16), _f32)),
    )(h_in, wt, b2, af_src, af_tgt, g2)

    z16 = jnp.zeros((NP, 16), _f32)
    z128 = jnp.zeros((NP, D), _f32)

    expe, dp = _pass1(src_p, tgt_p, ss, st, z16)

    rd = pl.pallas_call(
        _rdenom_body,
        out_shape=jax.ShapeDtypeStruct((NP, 16), _f32),
    )(dp)

    op = _pass2(src_p2, tgt_p2, hp, rd, expe, z128)

    out = pl.pallas_call(
        _combine_body,
        out_shape=jax.ShapeDtypeStruct((N, D), _f32),
    )(op)
    return out


# speedup vs baseline: 82.0151x; 1.3776x over previous
import functools

import jax
import jax.numpy as jnp
from jax import lax
from jax.experimental import pallas as pl
from jax.experimental.pallas import tpu as pltpu
from jax.experimental.pallas import tpu_sc as plsc

N = 10000
E = 320000
F_IN = 128
F_OUT = 16
H = 8
D = H * F_OUT

NP = 10112
NTILES = 32
C2 = 64
CPT2 = 160
PHASES = 2
PER_PHASE = CPT2 // PHASES
EDGES_PER_TILE = C2 * CPT2
EP = EDGES_PER_TILE * NTILES
ROWS_PER_TILE = NP // 16

_f32 = jnp.float32

_GDN = lax.GatherDimensionNumbers(offset_dims=(), collapsed_slice_dims=(0,),
                                  start_index_map=(0,))


def _bcast(vec, h):
    idx = jnp.full((16, 1), h, jnp.int32)
    return lax.gather(vec, idx, _GDN, slice_sizes=(1,),
                      mode=lax.GatherScatterMode.PROMISE_IN_BOUNDS)


def _proj_body(h_ref, wt_ref, b_ref, af_src_ref, af_tgt_ref, g2_ref,
               hp_ref, ss_ref, st_ref):
    hp = jnp.dot(h_ref[...], wt_ref[...],
                 preferred_element_type=_f32) + b_ref[...]
    hp_ref[0:N, :] = hp
    hp_ref[N:NP, :] = jnp.zeros((NP - N, D), _f32)
    g2 = g2_ref[...]
    ss_ref[0:N, :] = jnp.dot(hp * af_src_ref[...], g2,
                             preferred_element_type=_f32)
    st_ref[0:N, :] = jnp.dot(hp * af_tgt_ref[...], g2,
                             preferred_element_type=_f32)
    pad = jnp.full((NP - N, 16), -1e30, _f32)
    ss_ref[N:NP, :] = pad
    st_ref[N:NP, :] = pad


def _finish_body(dp_ref, op_ref, gt_ref, out_ref):
    r = 1.0 / (dp_ref[0, 0:N, :] + dp_ref[1, 0:N, :] + 1e-16)
    r128 = jnp.dot(r, gt_ref[...], preferred_element_type=_f32)
    out_ref[...] = (op_ref[0, 0:N, :] + op_ref[1, 0:N, :]) * r128


def _fused_body(src_hbm, tgt_hbm, ss_hbm, st_hbm, hp_hbm, z16_hbm,
                z128_hbm, dp_hbm, op_hbm,
                sidx_ph, tidx_ph, srows0, trows0, hrows0,
                srows1, trows1, hrows1, denom_sh, out_sh, sem0, sem1):
    c = lax.axis_index("c")
    s = lax.axis_index("s")
    wid = c * 16 + s
    rsl = pl.ds(s * ROWS_PER_TILE, ROWS_PER_TILE)
    pltpu.sync_copy(z16_hbm.at[rsl], denom_sh.at[rsl])
    pltpu.sync_copy(z128_hbm.at[rsl], out_sh.at[rsl])
    plsc.subcore_barrier()
    base_t = wid * EDGES_PER_TILE

    for ph in range(PHASES):
        isl = pl.ds(wid * CPT2 + ph * PER_PHASE, PER_PHASE)
        pltpu.sync_copy(src_hbm.at[isl], sidx_ph)
        pltpu.sync_copy(tgt_hbm.at[isl], tidx_ph)

        def _start(j, sr, tr, hr, sem):
            pltpu.async_copy(ss_hbm.at[sidx_ph.at[j]], sr, sem)
            pltpu.async_copy(st_hbm.at[tidx_ph.at[j]], tr, sem)
            pltpu.async_copy(hp_hbm.at[sidx_ph.at[j]], hr, sem)

        def _work(j, sr, tr, hr, sem):
            pltpu.make_async_copy(ss_hbm.at[sidx_ph.at[j]], sr, sem).wait()
            pltpu.make_async_copy(st_hbm.at[tidx_ph.at[j]], tr, sem).wait()
            pltpu.make_async_copy(hp_hbm.at[sidx_ph.at[j]], hr, sem).wait()

            @plsc.parallel_loop(0, C2, unroll=4)
            def _edges(i):
                w = sr[i] + tr[i]
                e = jnp.exp(jnp.maximum(w, w * 0.2))
                sr[i] = e
                for h in range(H):
                    sl = pl.ds(h * F_OUT, F_OUT)
                    hr[i, sl] = hr[i, sl] * _bcast(e, h)

            pltpu.sync_copy(sr, denom_sh.at[tidx_ph.at[j]], add=True)
            pltpu.sync_copy(hr, out_sh.at[tidx_ph.at[j]], add=True)

            @pl.when(j + 2 < PER_PHASE)
            def _next():
                _start(j + 2, sr, tr, hr, sem)

        _start(0, srows0, trows0, hrows0, sem0)
        _start(1, srows1, trows1, hrows1, sem1)

        @pl.loop(0, PER_PHASE, step=2)
        def _chunks(j):
            _work(j, srows0, trows0, hrows0, sem0)
            _work(j + 1, srows1, trows1, hrows1, sem1)

    plsc.subcore_barrier()
    pltpu.sync_copy(out_sh.at[rsl], op_hbm.at[c].at[rsl])

    @pl.when(s == 0)
    def _dump():
        pltpu.sync_copy(denom_sh, dp_hbm.at[c])


_MESH = plsc.VectorSubcoreMesh(core_axis_name="c", subcore_axis_name="s",
                               num_cores=2, num_subcores=16)

_SC_PARAMS = pltpu.CompilerParams(use_tc_tiling_on_sc=False,
                                  needs_layout_passes=False)

_fused = functools.partial(
    pl.kernel,
    out_type=(jax.ShapeDtypeStruct((2, NP, 16), _f32),
              jax.ShapeDtypeStruct((2, NP, D), _f32)),
    mesh=_MESH,
    scratch_types=[
        pltpu.VMEM((PER_PHASE, C2), jnp.int32),
        pltpu.VMEM((PER_PHASE, C2), jnp.int32),
        pltpu.VMEM((C2, 16), _f32),
        pltpu.VMEM((C2, 16), _f32),
        pltpu.VMEM((C2, D), _f32),
        pltpu.VMEM((C2, 16), _f32),
        pltpu.VMEM((C2, 16), _f32),
        pltpu.VMEM((C2, D), _f32),
        pltpu.VMEM_SHARED((NP, 16), _f32),
        pltpu.VMEM_SHARED((NP, D), _f32),
        pltpu.SemaphoreType.DMA,
        pltpu.SemaphoreType.DMA,
    ],
    compiler_params=_SC_PARAMS)(_fused_body)


def kernel(h_in, edge_index, W, b, a_src, a_tgt):
    src = edge_index[0].astype(jnp.int32)
    tgt = edge_index[1].astype(jnp.int32)
    pad = jnp.full((EP - E,), N, jnp.int32)
    src_p2 = jnp.concatenate([src, pad]).reshape(EP // C2, C2)
    tgt_p2 = jnp.concatenate([tgt, pad]).reshape(EP // C2, C2)

    wt = W.T
    b2 = b.reshape(1, D)
    af_src = a_src.reshape(1, D)
    af_tgt = a_tgt.reshape(1, D)
    j = jnp.arange(D) // F_OUT
    g = (j[:, None] == jnp.arange(H)[None, :]).astype(_f32)
    g2 = jnp.concatenate([g, g], axis=1)
    gt = jnp.concatenate([g, jnp.zeros((D, H), _f32)], axis=1).T

    hp, ss, st = pl.pallas_call(
        _proj_body,
        out_shape=(jax.ShapeDtypeStruct((NP, D), _f32),
                   jax.ShapeDtypeStruct((NP, 16), _f32),
                   jax.ShapeDtypeStruct((NP, 16), _f32)),
    )(h_in, wt, b2, af_src, af_tgt, g2)

    z16 = jnp.zeros((NP, 16), _f32)
    z128 = jnp.zeros((NP, D), _f32)

    dp, op = _fused(src_p2, tgt_p2, ss, st, hp, z16, z128)

    out = pl.pallas_call(
        _finish_body,
        out_shape=jax.ShapeDtypeStruct((N, D), _f32),
    )(dp, op, gt)
    return out


# speedup vs baseline: 82.2658x; 1.0031x over previous
import functools

import jax
import jax.numpy as jnp
from jax import lax
from jax.experimental import pallas as pl
from jax.experimental.pallas import tpu as pltpu
from jax.experimental.pallas import tpu_sc as plsc

N = 10000
E = 320000
F_IN = 128
F_OUT = 16
H = 8
D = H * F_OUT

NP = 10112
NTILES = 32
C2 = 64
CPT2 = 160
PHASES = 2
PER_PHASE = CPT2 // PHASES
EDGES_PER_TILE = C2 * CPT2
EP = EDGES_PER_TILE * NTILES
ROWS_PER_TILE = NP // 16

_f32 = jnp.float32

_GDN = lax.GatherDimensionNumbers(offset_dims=(), collapsed_slice_dims=(0,),
                                  start_index_map=(0,))


def _bcast(vec, h):
    idx = jnp.full((16, 1), h, jnp.int32)
    return lax.gather(vec, idx, _GDN, slice_sizes=(1,),
                      mode=lax.GatherScatterMode.PROMISE_IN_BOUNDS)


def _proj_body(h_ref, wt_ref, b_ref, af_src_ref, af_tgt_ref, g2_ref,
               hp_ref, ss_ref, st_ref):
    hp = jnp.dot(h_ref[...], wt_ref[...],
                 preferred_element_type=_f32) + b_ref[...]
    hp_ref[0:N, :] = hp
    hp_ref[N:NP, :] = jnp.zeros((NP - N, D), _f32)
    g2 = g2_ref[...]
    ss_ref[0:N, :] = jnp.dot(hp * af_src_ref[...], g2,
                             preferred_element_type=_f32)
    st_ref[0:N, :] = jnp.dot(hp * af_tgt_ref[...], g2,
                             preferred_element_type=_f32)
    pad = jnp.full((NP - N, 16), -1e30, _f32)
    ss_ref[N:NP, :] = pad
    st_ref[N:NP, :] = pad


def _finish_body(dp_ref, op_ref, gt_ref, out_ref):
    r = 1.0 / (dp_ref[0, 0:N, :] + dp_ref[1, 0:N, :] + 1e-16)
    r128 = jnp.dot(r, gt_ref[...], preferred_element_type=_f32)
    out_ref[...] = (op_ref[0, 0:N, :] + op_ref[1, 0:N, :]) * r128


def _fused_body(src_hbm, tgt_hbm, ss_hbm, st_hbm, hp_hbm, z16_hbm,
                z128_hbm, dp_hbm, op_hbm,
                sidx_ph, tidx_ph, srows0, trows0, hrows0,
                srows1, trows1, hrows1, denom_sh, out_sh, sem0, sem1,
                semsc):
    c = lax.axis_index("c")
    s = lax.axis_index("s")
    wid = c * 16 + s
    rsl = pl.ds(s * ROWS_PER_TILE, ROWS_PER_TILE)
    pltpu.sync_copy(z16_hbm.at[rsl], denom_sh.at[rsl])
    pltpu.sync_copy(z128_hbm.at[rsl], out_sh.at[rsl])
    plsc.subcore_barrier()
    base_t = wid * EDGES_PER_TILE

    for ph in range(PHASES):
        isl = pl.ds(wid * CPT2 + ph * PER_PHASE, PER_PHASE)
        pltpu.sync_copy(src_hbm.at[isl], sidx_ph)
        pltpu.sync_copy(tgt_hbm.at[isl], tidx_ph)

        def _start(j, sr, tr, hr, sem):
            pltpu.async_copy(ss_hbm.at[sidx_ph.at[j]], sr, sem)
            pltpu.async_copy(st_hbm.at[tidx_ph.at[j]], tr, sem)
            pltpu.async_copy(hp_hbm.at[sidx_ph.at[j]], hr, sem)

        def _work(j, sr, tr, hr, sem):
            pltpu.make_async_copy(ss_hbm.at[sidx_ph.at[j]], sr, sem).wait()
            pltpu.make_async_copy(st_hbm.at[tidx_ph.at[j]], tr, sem).wait()
            pltpu.make_async_copy(hp_hbm.at[sidx_ph.at[j]], hr, sem).wait()

            @plsc.parallel_loop(0, C2, unroll=4)
            def _edges(i):
                w = sr[i] + tr[i]
                e = jnp.exp(jnp.maximum(w, w * 0.2))
                sr[i] = e
                for h in range(H):
                    sl = pl.ds(h * F_OUT, F_OUT)
                    hr[i, sl] = hr[i, sl] * _bcast(e, h)

            d1 = pltpu.async_copy(hr, out_sh.at[tidx_ph.at[j]], semsc,
                                  add=True)
            d2 = pltpu.async_copy(sr, denom_sh.at[tidx_ph.at[j]], semsc,
                                  add=True)
            d1.wait()
            d2.wait()

            @pl.when(j + 2 < PER_PHASE)
            def _next():
                _start(j + 2, sr, tr, hr, sem)

        _start(0, srows0, trows0, hrows0, sem0)
        _start(1, srows1, trows1, hrows1, sem1)

        @pl.loop(0, PER_PHASE, step=2)
        def _chunks(j):
            _work(j, srows0, trows0, hrows0, sem0)
            _work(j + 1, srows1, trows1, hrows1, sem1)

    plsc.subcore_barrier()
    pltpu.sync_copy(out_sh.at[rsl], op_hbm.at[c].at[rsl])

    @pl.when(s == 0)
    def _dump():
        pltpu.sync_copy(denom_sh, dp_hbm.at[c])


_MESH = plsc.VectorSubcoreMesh(core_axis_name="c", subcore_axis_name="s",
                               num_cores=2, num_subcores=16)

_SC_PARAMS = pltpu.CompilerParams(use_tc_tiling_on_sc=False,
                                  needs_layout_passes=False)

_fused = functools.partial(
    pl.kernel,
    out_type=(jax.ShapeDtypeStruct((2, NP, 16), _f32),
              jax.ShapeDtypeStruct((2, NP, D), _f32)),
    mesh=_MESH,
    scratch_types=[
        pltpu.VMEM((PER_PHASE, C2), jnp.int32),
        pltpu.VMEM((PER_PHASE, C2), jnp.int32),
        pltpu.VMEM((C2, 16), _f32),
        pltpu.VMEM((C2, 16), _f32),
        pltpu.VMEM((C2, D), _f32),
        pltpu.VMEM((C2, 16), _f32),
        pltpu.VMEM((C2, 16), _f32),
        pltpu.VMEM((C2, D), _f32),
        pltpu.VMEM_SHARED((NP, 16), _f32),
        pltpu.VMEM_SHARED((NP, D), _f32),
        pltpu.SemaphoreType.DMA,
        pltpu.SemaphoreType.DMA,
        pltpu.SemaphoreType.DMA,
    ],
    compiler_params=_SC_PARAMS)(_fused_body)


def kernel(h_in, edge_index, W, b, a_src, a_tgt):
    src = edge_index[0].astype(jnp.int32)
    tgt = edge_index[1].astype(jnp.int32)
    pad = jnp.full((EP - E,), N, jnp.int32)
    src_p2 = jnp.concatenate([src, pad]).reshape(EP // C2, C2)
    tgt_p2 = jnp.concatenate([tgt, pad]).reshape(EP // C2, C2)

    wt = W.T
    b2 = b.reshape(1, D)
    af_src = a_src.reshape(1, D)
    af_tgt = a_tgt.reshape(1, D)
    j = jnp.arange(D) // F_OUT
    g = (j[:, None] == jnp.arange(H)[None, :]).astype(_f32)
    g2 = jnp.concatenate([g, g], axis=1)
    gt = jnp.concatenate([g, jnp.zeros((D, H), _f32)], axis=1).T

    hp, ss, st = pl.pallas_call(
        _proj_body,
        out_shape=(jax.ShapeDtypeStruct((NP, D), _f32),
                   jax.ShapeDtypeStruct((NP, 16), _f32),
                   jax.ShapeDtypeStruct((NP, 16), _f32)),
    )(h_in, wt, b2, af_src, af_tgt, g2)

    z16 = jnp.zeros((NP, 16), _f32)
    z128 = jnp.zeros((NP, D), _f32)

    dp, op = _fused(src_p2, tgt_p2, ss, st, hp, z16, z128)

    out = pl.pallas_call(
        _finish_body,
        out_shape=jax.ShapeDtypeStruct((N, D), _f32),
    )(dp, op, gt)
    return out
